# Initial kernel scaffold; baseline (speedup 1.0000x reference)
import jax
import jax.numpy as jnp
from jax.experimental import pallas as pl


def kernel(tab_tokens, img_tokens):
    raise NotImplementedError("write your pallas kernel here")


# speedup vs baseline: 179.0459x; 179.0459x over previous
import functools

import numpy as np
import jax
import jax.numpy as jnp
from jax.experimental import pallas as pl
from jax.experimental.pallas import tpu as pltpu

BINS = 32
BATCH = 64
T_TAB = 2048
T_IMG = 4096
M = T_TAB + T_IMG
FEAT = 128
BT = 256
G_TAB = T_TAB // BT
G_IMG = T_IMG // BT
G = G_TAB + G_IMG


def _mean_body(tab_ref, img_ref, x_ref, mm_ref, amin_ref, amax_ref):
    i = pl.program_id(0)
    s = jax.lax.cond(
        i < G_TAB,
        lambda: jnp.sum(tab_ref[...], axis=2),
        lambda: jnp.sum(img_ref[...], axis=2),
    ) * np.float32(1.0 / FEAT)
    x_ref[...] = s
    pmin = jnp.min(s)
    pmax = jnp.max(s)

    @pl.when(i == 0)
    def _():
        amin_ref[0, 0] = pmin
        amax_ref[0, 0] = pmax

    @pl.when(i > 0)
    def _():
        amin_ref[0, 0] = jnp.minimum(amin_ref[0, 0], pmin)
        amax_ref[0, 0] = jnp.maximum(amax_ref[0, 0], pmax)

    @pl.when(i == G - 1)
    def _():
        mm_ref[0, 0] = amin_ref[0, 0]
        mm_ref[0, 1] = amax_ref[0, 0]


def _means_and_minmax(tab_tokens, img_tokens, interpret=False):
    return pl.pallas_call(
        _mean_body,
        grid=(G,),
        in_specs=[
            pl.BlockSpec((BATCH, BT, FEAT),
                         lambda i: (0, jnp.minimum(i, G_TAB - 1), 0)),
            pl.BlockSpec((BATCH, BT, FEAT),
                         lambda i: (0, jnp.maximum(i - G_TAB, 0), 0)),
        ],
        out_specs=[
            pl.BlockSpec((BATCH, BT), lambda i: (0, i)),
            pl.BlockSpec(memory_space=pltpu.SMEM),
        ],
        out_shape=[
            jax.ShapeDtypeStruct((BATCH, M), jnp.float32),
            jax.ShapeDtypeStruct((1, 2), jnp.float32),
        ],
        scratch_shapes=[
            pltpu.SMEM((1, 1), jnp.float32),
            pltpu.SMEM((1, 1), jnp.float32),
        ],
        interpret=interpret,
    )(tab_tokens, img_tokens)


def _hist_body(x_ref, mm_ref, out_ref):
    x = x_ref[...]
    xmin = mm_ref[0, 0]
    xmax = mm_ref[0, 1]
    scale = np.float32(BINS) / (xmax - xmin)
    t = (x - xmin) * scale
    idx = jnp.clip(t.astype(jnp.int32), 0, BINS - 1)
    acc = jnp.zeros((1, M), jnp.float32)
    for b in range(BINS):
        k = jnp.sum((idx == b).astype(jnp.float32), axis=0,
                    keepdims=True)
        acc = acc + k * jnp.log(jnp.maximum(k, 1.0))
    out_ref[...] = np.float32(np.log(BATCH)) - acc * np.float32(1.0 / BATCH)


def _hist_entropy(x, mm, interpret=False):
    return pl.pallas_call(
        _hist_body,
        in_specs=[
            pl.BlockSpec((BATCH, M), lambda: (0, 0)),
            pl.BlockSpec(memory_space=pltpu.SMEM),
        ],
        out_specs=pl.BlockSpec((1, M), lambda: (0, 0)),
        out_shape=jax.ShapeDtypeStruct((1, M), jnp.float32),
        interpret=interpret,
    )(x, mm)


def _kernel_impl(tab_tokens, img_tokens, interpret=False):
    x, mm = _means_and_minmax(tab_tokens, img_tokens, interpret=interpret)
    out = _hist_entropy(x, mm, interpret=interpret)
    return out.reshape(M)


def kernel(tab_tokens, img_tokens):
    return _kernel_impl(tab_tokens, img_tokens)


# speedup vs baseline: 334.2448x; 1.8668x over previous
import functools

import numpy as np
import jax
import jax.numpy as jnp
from jax.experimental import pallas as pl
from jax.experimental.pallas import tpu as pltpu

BINS = 32
BATCH = 64
T_TAB = 2048
T_IMG = 4096
M = T_TAB + T_IMG
FEAT = 128
BT = 256
G_TAB = T_TAB // BT
G_IMG = T_IMG // BT
G = G_TAB + G_IMG


def _fused_body(tab_ref, img_ref, out_ref, x_s, mm_s):
    i = pl.program_id(0)

    def reduce_block(ref, col0):
        s = jnp.sum(ref[...], axis=2) * np.float32(1.0 / FEAT)
        x_s[:, pl.ds(pl.multiple_of(col0, BT), BT)] = s
        pmin = jnp.min(s)
        pmax = jnp.max(s)

        @pl.when(i == 0)
        def _():
            mm_s[0, 0] = pmin
            mm_s[0, 1] = pmax

        @pl.when(i > 0)
        def _():
            mm_s[0, 0] = jnp.minimum(mm_s[0, 0], pmin)
            mm_s[0, 1] = jnp.maximum(mm_s[0, 1], pmax)

    @pl.when(i < G_TAB)
    def _():
        reduce_block(tab_ref, i * BT)

    @pl.when((i >= G_TAB) & (i < G))
    def _():
        reduce_block(img_ref, T_TAB + (i - G_TAB) * BT)

    @pl.when(i == G)
    def _():
        x = x_s[...]
        xmin = mm_s[0, 0]
        xmax = mm_s[0, 1]
        scale = np.float32(BINS) / (xmax - xmin)
        t = (x - xmin) * scale
        idx = jnp.clip(t.astype(jnp.int32), 0, BINS - 1)
        acc = jnp.zeros((1, M), jnp.float32)
        for b in range(BINS):
            k = jnp.sum((idx == b).astype(jnp.float32), axis=0,
                        keepdims=True)
            acc = acc + k * jnp.log(jnp.maximum(k, 1.0))
        out_ref[...] = (np.float32(np.log(BATCH))
                        - acc * np.float32(1.0 / BATCH))


def _fused(tab_tokens, img_tokens, interpret=False):
    return pl.pallas_call(
        _fused_body,
        grid=(G + 1,),
        in_specs=[
            pl.BlockSpec((BATCH, BT, FEAT),
                         lambda i: (0, jnp.minimum(i, G_TAB - 1), 0)),
            pl.BlockSpec((BATCH, BT, FEAT),
                         lambda i: (0, jnp.clip(i - G_TAB, 0, G_IMG - 1), 0)),
        ],
        out_specs=pl.BlockSpec((1, M), lambda i: (0, 0)),
        out_shape=jax.ShapeDtypeStruct((1, M), jnp.float32),
        scratch_shapes=[
            pltpu.VMEM((BATCH, M), jnp.float32),
            pltpu.SMEM((1, 2), jnp.float32),
        ],
        interpret=interpret,
    )(tab_tokens, img_tokens)


def _kernel_impl(tab_tokens, img_tokens, interpret=False):
    out = _fused(tab_tokens, img_tokens, interpret=interpret)
    return out.reshape(M)


def kernel(tab_tokens, img_tokens):
    return _kernel_impl(tab_tokens, img_tokens)
